# baseline (speedup 1.0000x reference)
import jax
import jax.numpy as jnp
from jax.experimental import pallas as pl

_BLOCK_ROWS = 1024


def _copy_body(x_ref, o_ref):
    o_ref[...] = x_ref[...]


def kernel(x, embed_weight, embed_ind, mask, sampled):
    n, d = x.shape
    return pl.pallas_call(
        _copy_body,
        grid=(pl.cdiv(n, _BLOCK_ROWS),),
        in_specs=[pl.BlockSpec((_BLOCK_ROWS, d), lambda i: (i, 0))],
        out_specs=pl.BlockSpec((_BLOCK_ROWS, d), lambda i: (i, 0)),
        out_shape=jax.ShapeDtypeStruct((n, d), x.dtype),
    )(x)
